# baseline (speedup 1.0000x reference)
import functools

import jax
import jax.numpy as jnp
from jax import lax
from jax.experimental import pallas as pl
from jax.experimental.pallas import tpu as pltpu
from jax.experimental.pallas import tpu_sc as plsc

BATCH = 16384
EMB_DIM = 32
NC = 2
NS = 16
NW = NC * NS
BPW = BATCH // NW

NB = 7813
BPT = 245
SB = 6
NSUB = 41
NSB = 7
LCAP = 656
SBCAP = 160
MCAP = 48
MBLK = MCAP // 16
ISLOTS = 3 * BATCH + NW


def _p1_body(h_hbm, r_hbm, t_hbm, ent_t_hbm, rel_t_hbm, inter_hbm,
             bh_v, br_v, bt_v, jh_v, sh_v, jr_v, sr_v, jt_v, st_v,
             sbj_v, sbs_v, cnt_v, dbuf_v, stage_v, jsub_v, ssub_v, ridx_v,
             sem, ssem):
    wid = lax.axis_index("s") * NC + lax.axis_index("c")
    lo_blk = wid * BPT
    hi_blk = jnp.minimum(lo_blk + BPT, NB)
    lanes = lax.iota(jnp.int32, 16)
    dummy = 3 * BATCH + wid

    def fone(v, p, pos, ibuf, jl, sl, tau):
        j = ibuf[pl.ds(v * 16, 16)]
        jb = j >> 7
        mask = (jb >= lo_blk) & (jb < hi_blk)
        slot = jnp.full((16,), tau * BATCH + p * 4096, jnp.int32) \
            + v * 16 + lanes
        plsc.store_compressed(jl.at[pl.ds(pos, 16)], j, mask=mask)
        plsc.store_compressed(sl.at[pl.ds(pos, 16)], slot, mask=mask)
        cnt = plsc.all_reduce_population_count(mask)
        return pos + cnt[0]

    hp, rp, tp = 0, 0, 0
    for p in range(4):
        pltpu.sync_copy(h_hbm.at[pl.ds(p * 4096, 4096)], bh_v)
        pltpu.sync_copy(r_hbm.at[pl.ds(p * 4096, 4096)], br_v)
        pltpu.sync_copy(t_hbm.at[pl.ds(p * 4096, 4096)], bt_v)

        def fbody(v, c, p=p):
            a, b, d = c
            a = fone(v, p, a, bh_v, jh_v, sh_v, 0)
            d = fone(v, p, d, br_v, jr_v, sr_v, 1)
            b = fone(v, p, b, bt_v, jt_v, st_v, 2)
            return (a, b, d)

        hp, tp, rp = lax.fori_loop(0, 256, fbody, (hp, tp, rp))

    cvecs = [jnp.zeros((16,), jnp.int32) for _ in range(3)]
    for k in range(NSB):
        blo = lo_blk + k * (SB * SB)
        bhi = blo + SB * SB

        def sbone(v, pos, jl, sl, npos, row):
            jv = jl[pl.ds(v * 16, 16)]
            sv = sl[pl.ds(v * 16, 16)]
            jb = jv >> 7
            mask = ((v * 16 + lanes) < npos) & (jb >= blo) & (jb < bhi)
            plsc.store_compressed(sbj_v.at[pl.ds(row * SBCAP + pos, 16)], jv, mask=mask)
            plsc.store_compressed(sbs_v.at[pl.ds(row * SBCAP + pos, 16)], sv, mask=mask)
            cnt = plsc.all_reduce_population_count(mask)
            return pos + cnt[0]

        def sbody(v, c, k=k, blo=blo, bhi=bhi):
            a, b, d = c
            a = sbone(v, a, jh_v, sh_v, hp, 0 * NSB + k)
            b = sbone(v, b, jt_v, st_v, tp, 1 * NSB + k)
            d = sbone(v, d, jr_v, sr_v, rp, 2 * NSB + k)
            return (a, b, d)

        a, b, d = lax.fori_loop(0, LCAP // 16, sbody, (0, 0, 0))
        for ci, c in enumerate((a, b, d)):
            cvecs[ci] = jnp.where(lanes == k, c, cvecs[ci])
    for ci in range(3):
        cnt_v[ci, pl.ds(0, 16)] = cvecs[ci]

    def win_start(tab_hbm, s):
        c0 = jnp.minimum(lo_blk + s * SB, NB - SB) * 128
        pltpu.make_async_copy(
            tab_hbm.at[:, pl.ds(pl.multiple_of(c0, 128), SB * 128)],
            dbuf_v.at[pl.ds(pl.multiple_of((s % 2) * EMB_DIM, 32), EMB_DIM), :], sem,
        ).start()

    def win_wait():
        pltpu.make_async_copy(
            ent_t_hbm.at[:, pl.ds(0, SB * 128)], dbuf_v.at[pl.ds(0, EMB_DIM), :], sem,
        ).wait()

    def scat_wait():
        pltpu.make_async_copy(
            stage_v.at[pl.ds(0, 16), :],
            inter_hbm.at[ridx_v.at[0]], ssem,
        ).wait()

    def scan_table(tab_hbm, classes):
        nscat = len(classes) * MBLK

        def sub(s, _):
            @pl.when(s + 1 < NSUB)
            def _():
                win_start(tab_hbm, s + 1)
            win_wait()

            @pl.when(s >= 2)
            def _():
                for _x in range(nscat):
                    scat_wait()

            par = s % 2
            k = (s * 10923) >> 16
            c0 = jnp.minimum(lo_blk + s * SB, NB - SB)
            nlo = lo_blk + s * SB
            nhi = jnp.minimum(nlo + SB, hi_blk)
            parv = jnp.full((16,), par, jnp.int32)

            for ci, cls in enumerate(classes):
                row = cls * NSB + k
                npos = plsc.load_gather(
                    cnt_v, [jnp.full((16,), cls, jnp.int32),
                            jnp.full((16,), k, jnp.int32)])

                def l2(v, pos2, row=row, npos=npos):
                    jv = sbj_v[pl.ds(row * SBCAP + v * 16, 16)]
                    sv = sbs_v[pl.ds(row * SBCAP + v * 16, 16)]
                    jb = jv >> 7
                    mask = ((v * 16 + lanes) < npos) \
                        & (jb >= nlo) & (jb < nhi)
                    plsc.store_compressed(
                        jsub_v.at[pl.ds(pos2, 16)], jv, mask=mask)
                    plsc.store_compressed(
                        ssub_v.at[pl.ds(pos2, 16)], sv, mask=mask)
                    cnt = plsc.all_reduce_population_count(mask)
                    return pos2 + cnt[0]

                cnt = lax.fori_loop(0, SBCAP // 16, l2, 0)

                for m in range(MBLK):
                    sv = ssub_v[pl.ds(m * 16, 16)]
                    valid = (m * 16 + lanes) < cnt
                    crow = par * 2 * MBLK + ci * MBLK + m
                    ridx_v[crow, pl.ds(0, 16)] = jnp.where(
                        valid, sv, dummy)
                    jv = jsub_v[pl.ds(m * 16, 16)]
                    colv = jnp.where(
                        valid, ((jv >> 7) - c0) * 128 + (jv & 127), 0)
                    srow = (ci * MBLK + m) * 16

                    @pl.when(cnt > m * 16)
                    def _(m=m, colv=colv, srow=srow):
                        for i in range(16):
                            cs = jnp.full((16,), colv[i], jnp.int32)
                            p0 = plsc.load_gather(dbuf_v, [parv * EMB_DIM + lanes, cs])
                            p1 = plsc.load_gather(
                                dbuf_v, [parv * EMB_DIM + lanes + 16, cs])
                            stage_v[par * 2 * MCAP + srow + i, pl.ds(0, 16)] = p0
                            stage_v[par * 2 * MCAP + srow + i, pl.ds(16, 16)] = p1

                    pltpu.make_async_copy(
                        stage_v.at[pl.ds(pl.multiple_of(par * 2 * MCAP + srow, 16), 16), :],
                        inter_hbm.at[ridx_v.at[crow]], ssem,
                    ).start()
            return 0

        win_start(tab_hbm, 0)
        lax.fori_loop(0, NSUB, sub, 0)
        for _x in range(2 * nscat):
            scat_wait()

    scan_table(ent_t_hbm, (0, 1))
    scan_table(rel_t_hbm, (2,))


def _p2_body(inter_hbm, out_hbm, bh_v, br_v, bt_v, out_v, sem):
    wid = lax.axis_index("s") * NC + lax.axis_index("c")
    base = wid * BPW
    lanes = lax.iota(jnp.int32, 16)

    def chunk(b, _):
        row0 = base + b * 128
        pltpu.sync_copy(
            inter_hbm.at[pl.ds(pl.multiple_of(row0, 8), 128), :], bh_v)
        pltpu.sync_copy(
            inter_hbm.at[pl.ds(pl.multiple_of(BATCH + row0, 8), 128), :], br_v)
        pltpu.sync_copy(
            inter_hbm.at[pl.ds(pl.multiple_of(2 * BATCH + row0, 8), 128), :],
            bt_v)

        def group(g, _):
            acc = jnp.zeros((16,), jnp.float32)
            for i in range(16):
                r = g * 16 + i
                half = (bh_v[r, pl.ds(0, 16)] * br_v[r, pl.ds(0, 16)]
                        * bt_v[r, pl.ds(0, 16)]
                        + bh_v[r, pl.ds(16, 16)] * br_v[r, pl.ds(16, 16)]
                        * bt_v[r, pl.ds(16, 16)])
                acc = jnp.where(lanes == i, jnp.sum(half), acc)
            out_v[pl.ds(b * 128 + g * 16, 16)] = acc
            return 0

        lax.fori_loop(0, 8, group, 0)
        return 0

    lax.fori_loop(0, BPW // 128, chunk, 0)
    pltpu.sync_copy(out_v, out_hbm.at[pl.ds(base, BPW)])


_PARAMS = pltpu.CompilerParams(
    needs_layout_passes=False, use_tc_tiling_on_sc=True)


@jax.jit
def _distmult(hs, rs, ts, ent_t, rel_t):
    mesh = plsc.VectorSubcoreMesh(core_axis_name="c", subcore_axis_name="s")
    p1 = functools.partial(
        pl.kernel,
        mesh=mesh,
        compiler_params=_PARAMS,
        out_type=jax.ShapeDtypeStruct((ISLOTS, 128), jnp.float32),
        scratch_types=[
            pltpu.VMEM((4096,), jnp.int32),
            pltpu.VMEM((4096,), jnp.int32),
            pltpu.VMEM((4096,), jnp.int32),
            pltpu.VMEM((LCAP,), jnp.int32),
            pltpu.VMEM((LCAP,), jnp.int32),
            pltpu.VMEM((LCAP,), jnp.int32),
            pltpu.VMEM((LCAP,), jnp.int32),
            pltpu.VMEM((LCAP,), jnp.int32),
            pltpu.VMEM((LCAP,), jnp.int32),
            pltpu.VMEM((3 * NSB * SBCAP,), jnp.int32),
            pltpu.VMEM((3 * NSB * SBCAP,), jnp.int32),
            pltpu.VMEM((3, 16), jnp.int32),
            pltpu.VMEM((2 * EMB_DIM, SB * 128), jnp.float32),
            pltpu.VMEM((4 * MCAP, 128), jnp.float32),
            pltpu.VMEM((MCAP + 16,), jnp.int32),
            pltpu.VMEM((MCAP + 16,), jnp.int32),
            pltpu.VMEM((4 * MBLK, 16), jnp.int32),
            pltpu.SemaphoreType.DMA,
            pltpu.SemaphoreType.DMA,
        ],
    )(_p1_body)
    inter = p1(hs, rs, ts, ent_t, rel_t)

    p2 = functools.partial(
        pl.kernel,
        mesh=mesh,
        compiler_params=_PARAMS,
        out_type=jax.ShapeDtypeStruct((BATCH,), jnp.float32),
        scratch_types=[
            pltpu.VMEM((128, 128), jnp.float32),
            pltpu.VMEM((128, 128), jnp.float32),
            pltpu.VMEM((128, 128), jnp.float32),
            pltpu.VMEM((BPW,), jnp.float32),
            pltpu.SemaphoreType.DMA,
        ],
    )(_p2_body)
    return p2(inter)


def kernel(batch, ent_embs, rel_embs):
    hs = batch[:, 0]
    rs = batch[:, 1]
    ts = batch[:, 2]
    return _distmult(hs, rs, ts, ent_embs.T, rel_embs.T)


# speedup vs baseline: 1.3527x; 1.3527x over previous
import functools

import jax
import jax.numpy as jnp
from jax import lax
from jax.experimental import pallas as pl
from jax.experimental.pallas import tpu as pltpu
from jax.experimental.pallas import tpu_sc as plsc

BATCH = 16384
EMB_DIM = 32
NC = 2
NS = 16
NW = NC * NS
BPW = BATCH // NW
GRP = BPW // 16


def _distmult_body(h_hbm, r_hbm, t_hbm, ent_t_hbm, rel_t_hbm, out_hbm,
                   hidx_v, ridx_v, tidx_v, win_v, stage_v, out_v, sem):
    wid = lax.axis_index("s") * NC + lax.axis_index("c")
    base = wid * BPW

    pltpu.sync_copy(h_hbm.at[pl.ds(base, BPW)], hidx_v)
    pltpu.sync_copy(r_hbm.at[pl.ds(base, BPW)], ridx_v)
    pltpu.sync_copy(t_hbm.at[pl.ds(base, BPW)], tidx_v)

    lanes = lax.iota(jnp.int32, 16)

    def make_phase(idx_v, tab_hbm, tau):
        def phase(g, _):
            jvec = idx_v[pl.ds(g * 16, 16)]
            for i in range(16):
                jb128 = (jvec[i] >> 7) * 128
                pltpu.make_async_copy(
                    tab_hbm.at[:, pl.ds(pl.multiple_of(jb128, 128), 128)],
                    win_v.at[i], sem,
                ).start()
            for i in range(16):
                pltpu.make_async_copy(
                    tab_hbm.at[:, pl.ds(0, 128)], win_v.at[i], sem,
                ).wait()
            mvec = jvec & 127
            for i in range(16):
                mv = jnp.full((16,), mvec[i], jnp.int32)
                sv = jnp.full((16,), i, jnp.int32)
                p0 = plsc.load_gather(win_v, [sv, lanes, mv])
                p1 = plsc.load_gather(win_v, [sv, lanes + 16, mv])
                off = (tau * BPW + g * 16 + i) * EMB_DIM
                stage_v[pl.ds(off, 16)] = p0
                stage_v[pl.ds(off + 16, 16)] = p1
            return 0
        return phase

    lax.fori_loop(0, GRP, make_phase(hidx_v, ent_t_hbm, 0), 0)
    lax.fori_loop(0, GRP, make_phase(ridx_v, rel_t_hbm, 1), 0)
    lax.fori_loop(0, GRP, make_phase(tidx_v, ent_t_hbm, 2), 0)

    def group(g, _):
        s = g * 16
        acc = jnp.zeros((16,), jnp.float32)
        for i in range(16):
            r = s + i
            h0 = stage_v[pl.ds(r * EMB_DIM, 16)]
            h1 = stage_v[pl.ds(r * EMB_DIM + 16, 16)]
            r0 = stage_v[pl.ds((BPW + r) * EMB_DIM, 16)]
            r1 = stage_v[pl.ds((BPW + r) * EMB_DIM + 16, 16)]
            t0 = stage_v[pl.ds((2 * BPW + r) * EMB_DIM, 16)]
            t1 = stage_v[pl.ds((2 * BPW + r) * EMB_DIM + 16, 16)]
            half = h0 * r0 * t0 + h1 * r1 * t1
            acc = jnp.where(lanes == i, jnp.sum(half), acc)
        out_v[pl.ds(s, 16)] = acc
        return 0

    lax.fori_loop(0, GRP, group, 0)

    pltpu.sync_copy(out_v, out_hbm.at[pl.ds(base, BPW)])


@jax.jit
def _distmult(hs, rs, ts, ent_t, rel_t):
    mesh = plsc.VectorSubcoreMesh(core_axis_name="c", subcore_axis_name="s")
    kern = functools.partial(
        pl.kernel,
        mesh=mesh,
        compiler_params=pltpu.CompilerParams(
            needs_layout_passes=False, use_tc_tiling_on_sc=True),
        out_type=jax.ShapeDtypeStruct((BATCH,), jnp.float32),
        scratch_types=[
            pltpu.VMEM((BPW,), jnp.int32),
            pltpu.VMEM((BPW,), jnp.int32),
            pltpu.VMEM((BPW,), jnp.int32),
            pltpu.VMEM((16, EMB_DIM, 128), jnp.float32),
            pltpu.VMEM((3 * BPW * EMB_DIM,), jnp.float32),
            pltpu.VMEM((BPW,), jnp.float32),
            pltpu.SemaphoreType.DMA,
        ],
    )(_distmult_body)
    return kern(hs, rs, ts, ent_t, rel_t)


def kernel(batch, ent_embs, rel_embs):
    hs = batch[:, 0]
    rs = batch[:, 1]
    ts = batch[:, 2]
    return _distmult(hs, rs, ts, ent_embs.T, rel_embs.T)


# speedup vs baseline: 1.3710x; 1.0135x over previous
import functools

import jax
import jax.numpy as jnp
from jax import lax
from jax.experimental import pallas as pl
from jax.experimental.pallas import tpu as pltpu
from jax.experimental.pallas import tpu_sc as plsc

BATCH = 16384
EMB_DIM = 32
NC = 2
NS = 16
NW = NC * NS
BPW = BATCH // NW
GRP = BPW // 16


def _distmult_body(h_hbm, r_hbm, t_hbm, ent_t_hbm, rel_t_hbm, out_hbm,
                   hidx_v, ridx_v, tidx_v, win_v, stage_v, out_v,
                   sem_a, sem_b):
    wid = lax.axis_index("s") * NC + lax.axis_index("c")
    base = wid * BPW

    pltpu.sync_copy(h_hbm.at[pl.ds(base, BPW)], hidx_v)
    pltpu.sync_copy(r_hbm.at[pl.ds(base, BPW)], ridx_v)
    pltpu.sync_copy(t_hbm.at[pl.ds(base, BPW)], tidx_v)

    lanes = lax.iota(jnp.int32, 16)

    def run_phase(idx_v, tab_hbm, tau):
        def issue8(jvec, half, bank, sem):
            for i in range(8):
                jb128 = (jvec[half * 8 + i] >> 7) * 128
                pltpu.make_async_copy(
                    tab_hbm.at[:, pl.ds(pl.multiple_of(jb128, 128), 128)],
                    win_v.at[bank * 8 + i], sem,
                ).start()

        def wait8(sem):
            for _i in range(8):
                pltpu.make_async_copy(
                    tab_hbm.at[:, pl.ds(0, 128)], win_v.at[0], sem,
                ).wait()

        def extract8(mvec, half, bank, g):
            for i in range(8):
                mv = jnp.full((16,), mvec[half * 8 + i], jnp.int32)
                sv = jnp.full((16,), bank * 8 + i, jnp.int32)
                p0 = plsc.load_gather(win_v, [sv, lanes, mv])
                p1 = plsc.load_gather(win_v, [sv, lanes + 16, mv])
                off = (tau * BPW + g * 16 + half * 8 + i) * EMB_DIM
                stage_v[pl.ds(off, 16)] = p0
                stage_v[pl.ds(off + 16, 16)] = p1

        jvec0 = idx_v[pl.ds(0, 16)]
        issue8(jvec0, 0, 0, sem_a)

        def body(g, jvec):
            issue8(jvec, 1, 1, sem_b)
            wait8(sem_a)
            mvec = jvec & 127
            extract8(mvec, 0, 0, g)
            gn = jnp.minimum((g + 1) * 16, BPW - 16)
            jnext = idx_v[pl.ds(gn, 16)]
            issue8(jnext, 0, 0, sem_a)
            wait8(sem_b)
            extract8(mvec, 1, 1, g)
            return jnext

        lax.fori_loop(0, GRP, body, jvec0)
        wait8(sem_a)

    run_phase(hidx_v, ent_t_hbm, 0)
    run_phase(ridx_v, rel_t_hbm, 1)
    run_phase(tidx_v, ent_t_hbm, 2)

    def group(g, _):
        s = g * 16
        acc = jnp.zeros((16,), jnp.float32)
        for i in range(16):
            r = s + i
            h0 = stage_v[pl.ds(r * EMB_DIM, 16)]
            h1 = stage_v[pl.ds(r * EMB_DIM + 16, 16)]
            r0 = stage_v[pl.ds((BPW + r) * EMB_DIM, 16)]
            r1 = stage_v[pl.ds((BPW + r) * EMB_DIM + 16, 16)]
            t0 = stage_v[pl.ds((2 * BPW + r) * EMB_DIM, 16)]
            t1 = stage_v[pl.ds((2 * BPW + r) * EMB_DIM + 16, 16)]
            half = h0 * r0 * t0 + h1 * r1 * t1
            acc = jnp.where(lanes == i, jnp.sum(half), acc)
        out_v[pl.ds(s, 16)] = acc
        return 0

    lax.fori_loop(0, GRP, group, 0)

    pltpu.sync_copy(out_v, out_hbm.at[pl.ds(base, BPW)])


@jax.jit
def _distmult(hs, rs, ts, ent_t, rel_t):
    mesh = plsc.VectorSubcoreMesh(core_axis_name="c", subcore_axis_name="s")
    kern = functools.partial(
        pl.kernel,
        mesh=mesh,
        compiler_params=pltpu.CompilerParams(
            needs_layout_passes=False, use_tc_tiling_on_sc=True),
        out_type=jax.ShapeDtypeStruct((BATCH,), jnp.float32),
        scratch_types=[
            pltpu.VMEM((BPW,), jnp.int32),
            pltpu.VMEM((BPW,), jnp.int32),
            pltpu.VMEM((BPW,), jnp.int32),
            pltpu.VMEM((16, EMB_DIM, 128), jnp.float32),
            pltpu.VMEM((3 * BPW * EMB_DIM,), jnp.float32),
            pltpu.VMEM((BPW,), jnp.float32),
            pltpu.SemaphoreType.DMA,
            pltpu.SemaphoreType.DMA,
        ],
    )(_distmult_body)
    return kern(hs, rs, ts, ent_t, rel_t)


def kernel(batch, ent_embs, rel_embs):
    hs = batch[:, 0]
    rs = batch[:, 1]
    ts = batch[:, 2]
    return _distmult(hs, rs, ts, ent_embs.T, rel_embs.T)
